# Initial kernel scaffold; baseline (speedup 1.0000x reference)
import jax
import jax.numpy as jnp
from jax.experimental import pallas as pl


def kernel(hidden_states, weight, bias):
    raise NotImplementedError("write your pallas kernel here")


# speedup vs baseline: 5.1796x; 5.1796x over previous
import functools

import jax
import jax.numpy as jnp
from jax import lax
from jax.experimental import pallas as pl
from jax.experimental.pallas import tpu as pltpu

_TOKENS = 8192
_HIDDEN = 2048
_EXPERTS = 64
_K = 8
_BT = 512


def _router_body(x_ref, wt_ref, b_ref, scores_ref, idx_ref):
    x = x_ref[...]
    logits = jnp.dot(x, wt_ref[...], preferred_element_type=jnp.float32)
    logits = logits + b_ref[...]
    iota = lax.broadcasted_iota(jnp.int32, logits.shape, 1)
    work = logits
    sel_mask = jnp.zeros(logits.shape, dtype=jnp.bool_)
    idx_cols = []
    max0 = None
    for k in range(_K):
        m = jnp.max(work, axis=1, keepdims=True)
        if k == 0:
            max0 = m
        is_max = work == m
        idxk = jnp.min(jnp.where(is_max, iota, _EXPERTS), axis=1, keepdims=True)
        sel = iota == idxk
        sel_mask = jnp.logical_or(sel_mask, sel)
        work = jnp.where(sel, -jnp.inf, work)
        idx_cols.append(idxk)
    e = jnp.where(sel_mask, jnp.exp(logits - max0), 0.0)
    s = jnp.sum(e, axis=1, keepdims=True)
    scores_ref[...] = e / s
    idx_ref[...] = jnp.concatenate(idx_cols, axis=1)


@jax.jit
def kernel(hidden_states, weight, bias):
    x = hidden_states.reshape(-1, _HIDDEN)
    wt = weight.T
    b = bias.reshape(1, _EXPERTS)
    grid = (_TOKENS // _BT,)
    scores, idx = pl.pallas_call(
        _router_body,
        grid=grid,
        in_specs=[
            pl.BlockSpec((_BT, _HIDDEN), lambda i: (i, 0)),
            pl.BlockSpec((_HIDDEN, _EXPERTS), lambda i: (0, 0)),
            pl.BlockSpec((1, _EXPERTS), lambda i: (0, 0)),
        ],
        out_specs=[
            pl.BlockSpec((_BT, _EXPERTS), lambda i: (i, 0)),
            pl.BlockSpec((_BT, _K), lambda i: (i, 0)),
        ],
        out_shape=[
            jax.ShapeDtypeStruct((_TOKENS, _EXPERTS), jnp.float32),
            jax.ShapeDtypeStruct((_TOKENS, _K), jnp.int32),
        ],
        compiler_params=pltpu.CompilerParams(
            dimension_semantics=("arbitrary",),
        ),
    )(x, wt, b)
    return (scores, idx)


# speedup vs baseline: 7.1716x; 1.3846x over previous
import jax
import jax.numpy as jnp
from jax import lax
from jax.experimental import pallas as pl
from jax.experimental.pallas import tpu as pltpu

_TOKENS = 8192
_HIDDEN = 2048
_EXPERTS = 64
_K = 8
_BT = 512


def _router_body(x_ref, wt_ref, b_ref, scores_ref, idx_ref):
    x = x_ref[...]
    logits = jnp.dot(x, wt_ref[...], preferred_element_type=jnp.float32)
    logits = logits + b_ref[...]

    iota = lax.broadcasted_iota(jnp.int32, logits.shape, 1)
    bits = lax.bitcast_convert_type(logits, jnp.int32)
    pos = logits >= 0.0
    low6 = jnp.where(pos, _EXPERTS - 1 - iota, iota)
    keys = lax.bitcast_convert_type(
        jnp.bitwise_or(jnp.bitwise_and(bits, ~(_EXPERTS - 1)), low6),
        jnp.float32,
    )

    neg_inf = jnp.float32(-jnp.inf)
    work = keys
    m_cols = []
    for _ in range(_K):
        m = jnp.max(work, axis=1, keepdims=True)
        work = jnp.where(work == m, neg_inf, work)
        m_cols.append(m)

    sel_mask = work == neg_inf
    e = jnp.where(sel_mask, jnp.exp(logits - m_cols[0]), 0.0)
    s = jnp.sum(e, axis=1, keepdims=True)
    scores_ref[...] = e / s

    mk = jnp.concatenate(m_cols, axis=1)
    mk_bits = lax.bitcast_convert_type(mk, jnp.int32)
    mk_low = jnp.bitwise_and(mk_bits, _EXPERTS - 1)
    idx_ref[...] = jnp.where(mk >= 0.0, _EXPERTS - 1 - mk_low, mk_low)


@jax.jit
def kernel(hidden_states, weight, bias):
    x = hidden_states.reshape(-1, _HIDDEN)
    wt = weight.T
    b = bias.reshape(1, _EXPERTS)
    grid = (_TOKENS // _BT,)
    scores, idx = pl.pallas_call(
        _router_body,
        grid=grid,
        in_specs=[
            pl.BlockSpec((_BT, _HIDDEN), lambda i: (i, 0)),
            pl.BlockSpec((_HIDDEN, _EXPERTS), lambda i: (0, 0)),
            pl.BlockSpec((1, _EXPERTS), lambda i: (0, 0)),
        ],
        out_specs=[
            pl.BlockSpec((_BT, _EXPERTS), lambda i: (i, 0)),
            pl.BlockSpec((_BT, _K), lambda i: (i, 0)),
        ],
        out_shape=[
            jax.ShapeDtypeStruct((_TOKENS, _EXPERTS), jnp.float32),
            jax.ShapeDtypeStruct((_TOKENS, _K), jnp.int32),
        ],
        compiler_params=pltpu.CompilerParams(
            dimension_semantics=("arbitrary",),
        ),
    )(x, wt, b)
    return (scores, idx)
